# baseline (speedup 1.0000x reference)
import functools

import jax
import jax.numpy as jnp
from jax import lax
from jax.experimental import pallas as pl
from jax.experimental.pallas import tpu as pltpu
from jax.experimental.pallas import tpu_sc as plsc

NUM_TOKENS = 1000000
EMBED_DIM = 32
B = 16384
L = 50

NW = 32
CHUNK = 1024
SUB = 128
NSUB = CHUNK // SUB
TOTAL = B * L
PER_W = TOTAL // NW
NCHUNK = PER_W // CHUNK


def _tanh16(v):
    a = jnp.abs(v)
    z = jnp.exp(a * -2.0)
    t = (1.0 - z) / (1.0 + z)
    return jnp.where(v < 0.0, -t, t)


def _sc_body(table_hbm, x_hbm, out_hbm, idx_v, rows_v, sem):
    wid = lax.axis_index("s") * 2 + lax.axis_index("c")

    def chunk_body(g, _):
        pltpu.sync_copy(x_hbm.at[wid, g], idx_v)
        copies = []
        for j in range(NSUB):
            copies.append(pltpu.async_copy(
                table_hbm.at[idx_v.at[j]],
                rows_v.at[pl.ds(j * SUB, SUB)],
                sem))
        for c in copies:
            c.wait()

        def row_body(r, _):
            for h in range(2):
                sl = pl.ds(h * 16, 16)
                rows_v[r, sl] = _tanh16(rows_v[r, sl])
            return 0

        lax.fori_loop(0, CHUNK, row_body, 0)
        pltpu.sync_copy(rows_v, out_hbm.at[wid, g])
        return 0

    lax.fori_loop(0, NCHUNK, chunk_body, 0)


@jax.jit
def kernel(x, table):
    x_r = x.reshape(NW, NCHUNK, NSUB, SUB)
    mesh = plsc.VectorSubcoreMesh(core_axis_name="c", subcore_axis_name="s")
    out = pl.kernel(
        _sc_body,
        out_type=jax.ShapeDtypeStruct((NW, NCHUNK, CHUNK, EMBED_DIM),
                                      jnp.float32),
        mesh=mesh,
        scratch_types=[
            pltpu.VMEM((NSUB, SUB), jnp.int32),
            pltpu.VMEM((CHUNK, EMBED_DIM), jnp.float32),
            pltpu.SemaphoreType.DMA,
        ],
        compiler_params=pltpu.CompilerParams(use_tc_tiling_on_sc=False),
    )(table, x_r)
    return out.reshape(B, L, EMBED_DIM)


# speedup vs baseline: 2.0887x; 2.0887x over previous
import functools

import jax
import jax.numpy as jnp
from jax import lax
from jax.experimental import pallas as pl
from jax.experimental.pallas import tpu as pltpu
from jax.experimental.pallas import tpu_sc as plsc

NUM_TOKENS = 1000000
EMBED_DIM = 32
B = 16384
L = 50

NW = 32
CHUNK = 512
SUB = 128
NSUB = CHUNK // SUB
TOTAL = B * L
PER_W = TOTAL // NW
NCHUNK = PER_W // CHUNK
NPAIR = NCHUNK // 2

BBR = 512
NGB = B // (4 * BBR)


def _fire(table_hbm, idx_v, rows_v, buf, sem):
    for j in range(NSUB):
        pltpu.async_copy(
            table_hbm.at[idx_v.at[buf, j]],
            rows_v.at[buf, pl.ds(j * SUB, SUB)],
            sem)


def _drain(table_hbm, idx_v, rows_v, buf, sem):
    for j in range(NSUB):
        pltpu.make_async_copy(
            table_hbm.at[idx_v.at[buf, j]],
            rows_v.at[buf, pl.ds(j * SUB, SUB)],
            sem).wait()


def _sc_body(table_hbm, x_hbm, out_hbm, idx_v, rows_v, sem0, sem1):
    wid = lax.axis_index("s") * 2 + lax.axis_index("c")
    base = wid * PER_W

    def store(buf, c):
        pltpu.sync_copy(rows_v.at[buf],
                        out_hbm.at[pl.ds(base + c * CHUNK, CHUNK)])

    pltpu.sync_copy(x_hbm.at[wid, 0], idx_v.at[0])
    _fire(table_hbm, idx_v, rows_v, 0, sem0)

    def pair(q, _):
        c1 = 2 * q + 1
        pltpu.sync_copy(x_hbm.at[wid, c1], idx_v.at[1])
        _fire(table_hbm, idx_v, rows_v, 1, sem1)
        _drain(table_hbm, idx_v, rows_v, 0, sem0)
        store(0, c1 - 1)

        @pl.when(q < NPAIR - 1)
        def _():
            pltpu.sync_copy(x_hbm.at[wid, c1 + 1], idx_v.at[0])
            _fire(table_hbm, idx_v, rows_v, 0, sem0)

        _drain(table_hbm, idx_v, rows_v, 1, sem1)
        store(1, c1)
        return 0

    lax.fori_loop(0, NPAIR, pair, 0)


def _sc_gather(table, x_r):
    mesh = plsc.VectorSubcoreMesh(core_axis_name="c", subcore_axis_name="s")
    return pl.kernel(
        _sc_body,
        out_type=jax.ShapeDtypeStruct((TOTAL, EMBED_DIM), jnp.float32),
        mesh=mesh,
        scratch_types=[
            pltpu.VMEM((2, NSUB, SUB), jnp.int32),
            pltpu.VMEM((2, CHUNK, EMBED_DIM), jnp.float32),
            pltpu.SemaphoreType.DMA,
            pltpu.SemaphoreType.DMA,
        ],
        compiler_params=pltpu.CompilerParams(use_tc_tiling_on_sc=False),
    )(table, x_r)


WT = 2048
BRT = WT // 4
NWT = -(-NUM_TOKENS // WT)
PAD_TOKENS = NWT * WT


def _tc0_body(in_ref, out_ref):
    x = in_ref[...]
    for m in range(4):
        v = x[:, m * BRT:(m + 1) * BRT]
        out_ref[:, m * EMBED_DIM:(m + 1) * EMBED_DIM] = v.T


def _tc_relayout_table(table_t):
    return pl.pallas_call(
        _tc0_body,
        grid=(NWT,),
        in_specs=[pl.BlockSpec((EMBED_DIM, WT), lambda w: (0, w))],
        out_specs=pl.BlockSpec((BRT, 128), lambda w: (w, 0)),
        out_shape=jax.ShapeDtypeStruct((PAD_TOKENS // 4, 128), jnp.float32),
    )(table_t)


def _tc_body(in_ref, out_ref):
    x = in_ref[...]
    for m in range(4):
        v = x[:, m * EMBED_DIM:(m + 1) * EMBED_DIM]
        out_ref[0, :, m * BBR:(m + 1) * BBR] = jnp.tanh(v).T


def _tc_tanh_t(rows2):
    return pl.pallas_call(
        _tc_body,
        grid=(L, NGB),
        in_specs=[pl.BlockSpec((BBR, 128), lambda l, g: (l * NGB + g, 0))],
        out_specs=pl.BlockSpec((1, EMBED_DIM, 4 * BBR), lambda l, g: (l, 0, g)),
        out_shape=jax.ShapeDtypeStruct((L, EMBED_DIM, B), jnp.float32),
    )(rows2)


@jax.jit
def kernel(x, table):
    x_p = (x.T.reshape(L, NGB, 4, BBR)
           .transpose(0, 1, 3, 2)
           .reshape(NW, NCHUNK, NSUB, SUB))
    rem = x_p % WT
    x_p = (x_p - rem) + 4 * (rem % BRT) + rem // BRT
    table_rm = _tc_relayout_table(table.T)
    rows = _sc_gather(table_rm.reshape(PAD_TOKENS, EMBED_DIM), x_p)
    t = _tc_tanh_t(rows.reshape(TOTAL * EMBED_DIM // 128, 128))
    return t.transpose(2, 0, 1)


# speedup vs baseline: 3.4661x; 1.6595x over previous
import functools

import jax
import jax.numpy as jnp
from jax import lax
from jax.experimental import pallas as pl
from jax.experimental.pallas import tpu as pltpu
from jax.experimental.pallas import tpu_sc as plsc

NUM_TOKENS = 1000000
EMBED_DIM = 32
B = 16384
L = 50

NW = 32
CHUNK = 512
SUB = 128
NSUB = CHUNK // SUB
TOTAL = B * L
PER_W = TOTAL // NW
NCHUNK = PER_W // CHUNK
NPAIR = NCHUNK // 2

BBR = 1024
NGB = B // (4 * BBR)


def _fire(table_hbm, idx_v, rows_v, buf, sem):
    for j in range(NSUB):
        pltpu.async_copy(
            table_hbm.at[idx_v.at[buf, j]],
            rows_v.at[buf, pl.ds(j * SUB, SUB)],
            sem)


def _drain(table_hbm, idx_v, rows_v, buf, sem):
    for j in range(NSUB):
        pltpu.make_async_copy(
            table_hbm.at[idx_v.at[buf, j]],
            rows_v.at[buf, pl.ds(j * SUB, SUB)],
            sem).wait()


def _sc_body(table_hbm, x_hbm, out_hbm, idx_v, rows_v, sem0, sem1):
    wid = lax.axis_index("s") * 2 + lax.axis_index("c")
    base = wid * PER_W

    def store(buf, c):
        pltpu.sync_copy(rows_v.at[buf],
                        out_hbm.at[pl.ds(base + c * CHUNK, CHUNK)])

    pltpu.sync_copy(x_hbm.at[wid, 0], idx_v.at[0])
    _fire(table_hbm, idx_v, rows_v, 0, sem0)

    def pair(q, _):
        c1 = 2 * q + 1
        pltpu.sync_copy(x_hbm.at[wid, c1], idx_v.at[1])
        _fire(table_hbm, idx_v, rows_v, 1, sem1)
        _drain(table_hbm, idx_v, rows_v, 0, sem0)
        store(0, c1 - 1)

        @pl.when(q < NPAIR - 1)
        def _():
            pltpu.sync_copy(x_hbm.at[wid, c1 + 1], idx_v.at[0])
            _fire(table_hbm, idx_v, rows_v, 0, sem0)

        _drain(table_hbm, idx_v, rows_v, 1, sem1)
        store(1, c1)
        return 0

    lax.fori_loop(0, NPAIR, pair, 0)


def _sc_gather(table, x_r):
    mesh = plsc.VectorSubcoreMesh(core_axis_name="c", subcore_axis_name="s")
    return pl.kernel(
        _sc_body,
        out_type=jax.ShapeDtypeStruct((TOTAL, EMBED_DIM), jnp.float32),
        mesh=mesh,
        scratch_types=[
            pltpu.VMEM((2, NSUB, SUB), jnp.int32),
            pltpu.VMEM((2, CHUNK, EMBED_DIM), jnp.float32),
            pltpu.SemaphoreType.DMA,
            pltpu.SemaphoreType.DMA,
        ],
        compiler_params=pltpu.CompilerParams(use_tc_tiling_on_sc=False),
    )(table, x_r)


WT = 4096
BRT = WT // 4
NWT = -(-NUM_TOKENS // WT)
PAD_TOKENS = NWT * WT


def _tc0_body(in_ref, out_ref, scratch):
    x = in_ref[...]
    for m in range(4):
        scratch[m * EMBED_DIM:(m + 1) * EMBED_DIM, :] = (
            x[:, m * BRT:(m + 1) * BRT])
    out_ref[...] = scratch[...].T


def _tc_relayout_table(table_t):
    return pl.pallas_call(
        _tc0_body,
        grid=(NWT,),
        in_specs=[pl.BlockSpec((EMBED_DIM, WT), lambda w: (0, w))],
        out_specs=pl.BlockSpec((BRT, 128), lambda w: (w, 0)),
        out_shape=jax.ShapeDtypeStruct((PAD_TOKENS // 4, 128), jnp.float32),
        scratch_shapes=[pltpu.VMEM((128, BRT), jnp.float32)],
    )(table_t)


def _tc_body(in_ref, out_ref):
    y = jnp.tanh(in_ref[...]).T
    for m in range(4):
        out_ref[0, :, m * BBR:(m + 1) * BBR] = (
            y[m * EMBED_DIM:(m + 1) * EMBED_DIM, :])


def _tc_tanh_t(rows2):
    return pl.pallas_call(
        _tc_body,
        grid=(L, NGB),
        in_specs=[pl.BlockSpec((BBR, 128), lambda l, g: (l * NGB + g, 0))],
        out_specs=pl.BlockSpec((1, EMBED_DIM, 4 * BBR), lambda l, g: (l, 0, g)),
        out_shape=jax.ShapeDtypeStruct((L, EMBED_DIM, B), jnp.float32),
    )(rows2)


@jax.jit
def kernel(x, table):
    x_p = (x.T.reshape(L, NGB, 4, BBR)
           .transpose(0, 1, 3, 2)
           .reshape(NW, NCHUNK, NSUB, SUB))
    rem = x_p & (WT - 1)
    x_p = (x_p - rem) + ((rem & (BRT - 1)) << 2) + (rem >> (BRT.bit_length() - 1))
    table_rm = _tc_relayout_table(table.T)
    rows = _sc_gather(table_rm.reshape(PAD_TOKENS, EMBED_DIM), x_p)
    t = _tc_tanh_t(rows.reshape(TOTAL * EMBED_DIM // 128, 128))
    return t.transpose(2, 0, 1)


# speedup vs baseline: 4.1689x; 1.2028x over previous
import functools

import jax
import jax.numpy as jnp
from jax import lax
from jax.experimental import pallas as pl
from jax.experimental.pallas import tpu as pltpu
from jax.experimental.pallas import tpu_sc as plsc

NUM_TOKENS = 1000000
EMBED_DIM = 32
B = 16384
L = 50

NW = 32
CHUNK = 512
SUB = 128
NSUB = CHUNK // SUB
TOTAL = B * L
PER_W = TOTAL // NW
NCHUNK = PER_W // CHUNK
NPAIR = NCHUNK // 2

BBR = 2048
NGB = B // (4 * BBR)


def _fire(table_hbm, idx_v, rows_v, buf, sem):
    for j in range(NSUB):
        pltpu.async_copy(
            table_hbm.at[idx_v.at[buf, j]],
            rows_v.at[buf, pl.ds(j * SUB, SUB)],
            sem)


def _drain(table_hbm, idx_v, rows_v, buf, sem):
    for j in range(NSUB):
        pltpu.make_async_copy(
            table_hbm.at[idx_v.at[buf, j]],
            rows_v.at[buf, pl.ds(j * SUB, SUB)],
            sem).wait()


def _sc_body(table_hbm, x_hbm, out_hbm, idx_v, rows_v, sem0, sem1):
    wid = lax.axis_index("s") * 2 + lax.axis_index("c")
    base = wid * PER_W

    def store(buf, c):
        pltpu.sync_copy(rows_v.at[buf],
                        out_hbm.at[pl.ds(base + c * CHUNK, CHUNK)])

    pltpu.sync_copy(x_hbm.at[wid, 0], idx_v.at[0])
    _fire(table_hbm, idx_v, rows_v, 0, sem0)

    def pair(q, _):
        c1 = 2 * q + 1
        pltpu.sync_copy(x_hbm.at[wid, c1], idx_v.at[1])
        _fire(table_hbm, idx_v, rows_v, 1, sem1)
        _drain(table_hbm, idx_v, rows_v, 0, sem0)
        store(0, c1 - 1)

        @pl.when(q < NPAIR - 1)
        def _():
            pltpu.sync_copy(x_hbm.at[wid, c1 + 1], idx_v.at[0])
            _fire(table_hbm, idx_v, rows_v, 0, sem0)

        _drain(table_hbm, idx_v, rows_v, 1, sem1)
        store(1, c1)
        return 0

    lax.fori_loop(0, NPAIR, pair, 0)


def _sc_gather(table, x_r):
    mesh = plsc.VectorSubcoreMesh(core_axis_name="c", subcore_axis_name="s")
    return pl.kernel(
        _sc_body,
        out_type=jax.ShapeDtypeStruct((TOTAL, EMBED_DIM), jnp.float32),
        mesh=mesh,
        scratch_types=[
            pltpu.VMEM((2, NSUB, SUB), jnp.int32),
            pltpu.VMEM((2, CHUNK, EMBED_DIM), jnp.float32),
            pltpu.SemaphoreType.DMA,
            pltpu.SemaphoreType.DMA,
        ],
        compiler_params=pltpu.CompilerParams(use_tc_tiling_on_sc=False),
    )(table, x_r)


WT = 8192
BRT = WT // 4
NWT = -(-NUM_TOKENS // WT)
PAD_TOKENS = NWT * WT


def _tc0_body(in_ref, out_ref, scratch):
    x = in_ref[...]
    for m in range(4):
        scratch[m * EMBED_DIM:(m + 1) * EMBED_DIM, :] = (
            x[:, m * BRT:(m + 1) * BRT])
    out_ref[...] = scratch[...].T


def _tc_relayout_table(table_t):
    return pl.pallas_call(
        _tc0_body,
        grid=(NWT,),
        in_specs=[pl.BlockSpec((EMBED_DIM, WT), lambda w: (0, w))],
        out_specs=pl.BlockSpec((BRT, 128), lambda w: (w, 0)),
        out_shape=jax.ShapeDtypeStruct((PAD_TOKENS // 4, 128), jnp.float32),
        scratch_shapes=[pltpu.VMEM((128, BRT), jnp.float32)],
    )(table_t)


def _tc_body(in_ref, out_ref):
    y = jnp.tanh(in_ref[...]).T
    for m in range(4):
        out_ref[0, :, m * BBR:(m + 1) * BBR] = (
            y[m * EMBED_DIM:(m + 1) * EMBED_DIM, :])


def _tc_tanh_t(rows2):
    return pl.pallas_call(
        _tc_body,
        grid=(L, NGB),
        in_specs=[pl.BlockSpec((BBR, 128), lambda l, g: (l * NGB + g, 0))],
        out_specs=pl.BlockSpec((1, EMBED_DIM, 4 * BBR), lambda l, g: (l, 0, g)),
        out_shape=jax.ShapeDtypeStruct((L, EMBED_DIM, B), jnp.float32),
    )(rows2)


@jax.jit
def kernel(x, table):
    x_p = (x.T.reshape(L, NGB, 4, BBR)
           .transpose(0, 1, 3, 2)
           .reshape(NW, NCHUNK, NSUB, SUB))
    rem = x_p & (WT - 1)
    x_p = (x_p - rem) + ((rem & (BRT - 1)) << 2) + (rem >> (BRT.bit_length() - 1))
    table_rm = _tc_relayout_table(table.T)
    rows = _sc_gather(table_rm.reshape(PAD_TOKENS, EMBED_DIM), x_p)
    t = _tc_tanh_t(rows.reshape(TOTAL * EMBED_DIM // 128, 128))
    return t.transpose(2, 0, 1)


# speedup vs baseline: 4.8229x; 1.1569x over previous
import functools

import jax
import jax.numpy as jnp
from jax import lax
from jax.experimental import pallas as pl
from jax.experimental.pallas import tpu as pltpu
from jax.experimental.pallas import tpu_sc as plsc

NUM_TOKENS = 1000000
EMBED_DIM = 32
B = 16384
L = 50

NW = 32
CHUNK = 512
SUB = 128
NSUB = CHUNK // SUB
TOTAL = B * L
PER_W = TOTAL // NW
NCHUNK = PER_W // CHUNK
NPAIR = NCHUNK // 2

BBR = 4096
NGB = B // (4 * BBR)


def _fire(table_hbm, idx_v, rows_v, buf, sem):
    for j in range(NSUB):
        pltpu.async_copy(
            table_hbm.at[idx_v.at[buf, j]],
            rows_v.at[buf, pl.ds(j * SUB, SUB)],
            sem)


def _drain(table_hbm, idx_v, rows_v, buf, sem):
    for j in range(NSUB):
        pltpu.make_async_copy(
            table_hbm.at[idx_v.at[buf, j]],
            rows_v.at[buf, pl.ds(j * SUB, SUB)],
            sem).wait()


def _sc_body(table_hbm, x_hbm, out_hbm, idx_v, rows_v, sem0, sem1):
    wid = lax.axis_index("s") * 2 + lax.axis_index("c")
    base = wid * PER_W

    def store(buf, c):
        pltpu.sync_copy(rows_v.at[buf],
                        out_hbm.at[pl.ds(base + c * CHUNK, CHUNK)])

    pltpu.sync_copy(x_hbm.at[wid, 0], idx_v.at[0])
    _fire(table_hbm, idx_v, rows_v, 0, sem0)

    def pair(q, _):
        c1 = 2 * q + 1
        pltpu.sync_copy(x_hbm.at[wid, c1], idx_v.at[1])
        _fire(table_hbm, idx_v, rows_v, 1, sem1)
        _drain(table_hbm, idx_v, rows_v, 0, sem0)
        store(0, c1 - 1)

        @pl.when(q < NPAIR - 1)
        def _():
            pltpu.sync_copy(x_hbm.at[wid, c1 + 1], idx_v.at[0])
            _fire(table_hbm, idx_v, rows_v, 0, sem0)

        _drain(table_hbm, idx_v, rows_v, 1, sem1)
        store(1, c1)
        return 0

    lax.fori_loop(0, NPAIR, pair, 0)


def _sc_gather(table, x_r):
    mesh = plsc.VectorSubcoreMesh(core_axis_name="c", subcore_axis_name="s")
    return pl.kernel(
        _sc_body,
        out_type=jax.ShapeDtypeStruct((TOTAL, EMBED_DIM), jnp.float32),
        mesh=mesh,
        scratch_types=[
            pltpu.VMEM((2, NSUB, SUB), jnp.int32),
            pltpu.VMEM((2, CHUNK, EMBED_DIM), jnp.float32),
            pltpu.SemaphoreType.DMA,
            pltpu.SemaphoreType.DMA,
        ],
        compiler_params=pltpu.CompilerParams(use_tc_tiling_on_sc=False),
    )(table, x_r)


WT = 16384
BRT = WT // 4
NWT = -(-NUM_TOKENS // WT)
PAD_TOKENS = NWT * WT


def _tc0_body(in_ref, out_ref, scratch):
    x = in_ref[...]
    for m in range(4):
        scratch[m * EMBED_DIM:(m + 1) * EMBED_DIM, :] = (
            x[:, m * BRT:(m + 1) * BRT])
    out_ref[...] = scratch[...].T


def _tc_relayout_table(table_t):
    return pl.pallas_call(
        _tc0_body,
        grid=(NWT,),
        in_specs=[pl.BlockSpec((EMBED_DIM, WT), lambda w: (0, w))],
        out_specs=pl.BlockSpec((BRT, 128), lambda w: (w, 0)),
        out_shape=jax.ShapeDtypeStruct((PAD_TOKENS // 4, 128), jnp.float32),
        scratch_shapes=[pltpu.VMEM((128, BRT), jnp.float32)],
    )(table_t)


def _tc_body(in_ref, out_ref):
    y = jnp.tanh(in_ref[...]).T
    for m in range(4):
        out_ref[0, :, m * BBR:(m + 1) * BBR] = (
            y[m * EMBED_DIM:(m + 1) * EMBED_DIM, :])


def _tc_tanh_t(rows2):
    return pl.pallas_call(
        _tc_body,
        grid=(L, NGB),
        in_specs=[pl.BlockSpec((BBR, 128), lambda l, g: (l * NGB + g, 0))],
        out_specs=pl.BlockSpec((1, EMBED_DIM, 4 * BBR), lambda l, g: (l, 0, g)),
        out_shape=jax.ShapeDtypeStruct((L, EMBED_DIM, B), jnp.float32),
    )(rows2)


@jax.jit
def kernel(x, table):
    x_p = (x.T.reshape(L, NGB, 4, BBR)
           .transpose(0, 1, 3, 2)
           .reshape(NW, NCHUNK, NSUB, SUB))
    rem = x_p & (WT - 1)
    x_p = (x_p - rem) + ((rem & (BRT - 1)) << 2) + (rem >> (BRT.bit_length() - 1))
    table_rm = _tc_relayout_table(table.T)
    rows = _sc_gather(table_rm.reshape(PAD_TOKENS, EMBED_DIM), x_p)
    t = _tc_tanh_t(rows.reshape(TOTAL * EMBED_DIM // 128, 128))
    return t.transpose(2, 0, 1)


# speedup vs baseline: 6.8865x; 1.4279x over previous
import functools

import jax
import jax.numpy as jnp
from jax import lax
from jax.experimental import pallas as pl
from jax.experimental.pallas import tpu as pltpu
from jax.experimental.pallas import tpu_sc as plsc

NUM_TOKENS = 1000000
EMBED_DIM = 32
B = 16384
L = 50

NW = 32
CHUNK = 512
SUB = 128
NSUB = CHUNK // SUB
TOTAL = B * L
PER_W = TOTAL // NW
NCHUNK = PER_W // CHUNK
NPAIR = NCHUNK // 2

BBR = 4096
NGB = B // (4 * BBR)


def _fire(table_hbm, idx_v, rows_v, buf, sem):
    for j in range(NSUB):
        pltpu.async_copy(
            table_hbm.at[idx_v.at[buf, pl.ds(j * SUB, SUB)]],
            rows_v.at[buf, pl.ds(j * SUB, SUB)],
            sem)


def _drain(table_hbm, idx_v, rows_v, buf, sem):
    for j in range(NSUB):
        pltpu.make_async_copy(
            table_hbm.at[idx_v.at[buf, pl.ds(j * SUB, SUB)]],
            rows_v.at[buf, pl.ds(j * SUB, SUB)],
            sem).wait()


def _sc_body(table_hbm, x_hbm, out_hbm, idx_v, rows_v, sem0, sem1):
    wid = lax.axis_index("s") * 2 + lax.axis_index("c")
    base = wid * PER_W

    def store(buf, c):
        p0 = base + c * CHUNK
        li = p0 >> 14
        r = p0 & (B - 1)
        m = r >> 12
        k0 = r & (B // 4 - 1)
        pltpu.sync_copy(rows_v.at[buf],
                        out_hbm.at[li, pl.ds(k0, CHUNK), m])

    pltpu.sync_copy(x_hbm.at[wid, 0], idx_v.at[0])
    _fire(table_hbm, idx_v, rows_v, 0, sem0)

    def pair(q, _):
        c1 = 2 * q + 1
        pltpu.sync_copy(x_hbm.at[wid, c1], idx_v.at[1])
        _fire(table_hbm, idx_v, rows_v, 1, sem1)
        _drain(table_hbm, idx_v, rows_v, 0, sem0)
        store(0, c1 - 1)

        @pl.when(q < NPAIR - 1)
        def _():
            pltpu.sync_copy(x_hbm.at[wid, c1 + 1], idx_v.at[0])
            _fire(table_hbm, idx_v, rows_v, 0, sem0)

        _drain(table_hbm, idx_v, rows_v, 1, sem1)
        store(1, c1)
        return 0

    lax.fori_loop(0, NPAIR, pair, 0)


def _sc_gather(table, x_r):
    mesh = plsc.VectorSubcoreMesh(core_axis_name="c", subcore_axis_name="s")
    return pl.kernel(
        _sc_body,
        out_type=jax.ShapeDtypeStruct((L, B // 4, 4, EMBED_DIM), jnp.float32),
        mesh=mesh,
        scratch_types=[
            pltpu.VMEM((2, CHUNK), jnp.int32),
            pltpu.VMEM((2, CHUNK, EMBED_DIM), jnp.float32),
            pltpu.SemaphoreType.DMA,
            pltpu.SemaphoreType.DMA,
        ],
        compiler_params=pltpu.CompilerParams(use_tc_tiling_on_sc=False),
    )(table, x_r)


WT = 16384
BRT = WT // 4
NWT = -(-NUM_TOKENS // WT)
PAD_TOKENS = NWT * WT


def _tc0_body(in_ref, out_ref, scratch):
    x = in_ref[...]
    for m in range(4):
        scratch[m * EMBED_DIM:(m + 1) * EMBED_DIM, :] = (
            x[:, m * BRT:(m + 1) * BRT])
    out_ref[...] = scratch[...].T


def _tc_relayout_table(table_t):
    return pl.pallas_call(
        _tc0_body,
        grid=(NWT,),
        in_specs=[pl.BlockSpec((EMBED_DIM, WT), lambda w: (0, w))],
        out_specs=pl.BlockSpec((BRT, 128), lambda w: (w, 0)),
        out_shape=jax.ShapeDtypeStruct((PAD_TOKENS // 4, 128), jnp.float32),
        scratch_shapes=[pltpu.VMEM((128, BRT), jnp.float32)],
    )(table_t)


def _tc_body(in_ref, out_ref):
    y = jnp.tanh(in_ref[...]).T
    for m in range(4):
        out_ref[0, :, m * BBR:(m + 1) * BBR] = (
            y[m * EMBED_DIM:(m + 1) * EMBED_DIM, :])


def _tc_tanh_t(rows2):
    return pl.pallas_call(
        _tc_body,
        grid=(L, NGB),
        in_specs=[pl.BlockSpec((BBR, 128), lambda l, g: (l * NGB + g, 0))],
        out_specs=pl.BlockSpec((1, EMBED_DIM, 4 * BBR), lambda l, g: (l, 0, g)),
        out_shape=jax.ShapeDtypeStruct((L, EMBED_DIM, B), jnp.float32),
    )(rows2)


@jax.jit
def kernel(x, table):
    x_p = x.T.reshape(NW, NCHUNK, CHUNK)
    rem = x_p & (WT - 1)
    x_p = (x_p - rem) + ((rem & (BRT - 1)) << 2) + (rem >> (BRT.bit_length() - 1))
    table_rm = _tc_relayout_table(table.T)
    rows = _sc_gather(table_rm.reshape(PAD_TOKENS, EMBED_DIM), x_p)
    t = _tc_tanh_t(rows.reshape(TOTAL * EMBED_DIM // 128, 128))
    return t.transpose(2, 0, 1)


# speedup vs baseline: 7.3687x; 1.0700x over previous
import functools

import jax
import jax.numpy as jnp
from jax import lax
from jax.experimental import pallas as pl
from jax.experimental.pallas import tpu as pltpu
from jax.experimental.pallas import tpu_sc as plsc

NUM_TOKENS = 1000000
EMBED_DIM = 32
B = 16384
L = 50

NW = 32
CHUNK = 512
SUB = 128
NSUB = CHUNK // SUB
TOTAL = B * L
LH = L // 2
TOTAL_H = B * LH
PER_W = TOTAL_H // NW
NCHUNK = PER_W // CHUNK
NPAIR = NCHUNK // 2

BBR = 4096
NGB = B // (4 * BBR)


def _fire(table_hbm, idx_v, rows_v, buf, sem):
    for j in range(NSUB):
        pltpu.async_copy(
            table_hbm.at[idx_v.at[buf, pl.ds(j * SUB, SUB)]],
            rows_v.at[buf, pl.ds(j * SUB, SUB)],
            sem)


def _drain(table_hbm, idx_v, rows_v, buf, sem):
    for j in range(NSUB):
        pltpu.make_async_copy(
            table_hbm.at[idx_v.at[buf, pl.ds(j * SUB, SUB)]],
            rows_v.at[buf, pl.ds(j * SUB, SUB)],
            sem).wait()


def _sc_body(table_hbm, x_hbm, out_hbm, idx_v, rows_v, sem0, sem1):
    wid = lax.axis_index("s") * 2 + lax.axis_index("c")
    base = wid * PER_W

    def store(buf, c):
        p0 = base + c * CHUNK
        li = p0 >> 14
        r = p0 & (B - 1)
        m = r >> 12
        k0 = r & (B // 4 - 1)
        pltpu.sync_copy(rows_v.at[buf],
                        out_hbm.at[li, pl.ds(k0, CHUNK), m])

    pltpu.sync_copy(x_hbm.at[wid, 0], idx_v.at[0])
    _fire(table_hbm, idx_v, rows_v, 0, sem0)

    def pair(q, _):
        c1 = 2 * q + 1
        pltpu.sync_copy(x_hbm.at[wid, c1], idx_v.at[1])
        _fire(table_hbm, idx_v, rows_v, 1, sem1)
        _drain(table_hbm, idx_v, rows_v, 0, sem0)
        store(0, c1 - 1)

        if NCHUNK % 2 == 1:
            pltpu.sync_copy(x_hbm.at[wid, c1 + 1], idx_v.at[0])
            _fire(table_hbm, idx_v, rows_v, 0, sem0)
        else:
            @pl.when(q < NPAIR - 1)
            def _():
                pltpu.sync_copy(x_hbm.at[wid, c1 + 1], idx_v.at[0])
                _fire(table_hbm, idx_v, rows_v, 0, sem0)

        _drain(table_hbm, idx_v, rows_v, 1, sem1)
        store(1, c1)
        return 0

    lax.fori_loop(0, NPAIR, pair, 0)
    if NCHUNK % 2 == 1:
        _drain(table_hbm, idx_v, rows_v, 0, sem0)
        store(0, NCHUNK - 1)


def _sc_gather(table, x_r):
    mesh = plsc.VectorSubcoreMesh(core_axis_name="c", subcore_axis_name="s")
    return pl.kernel(
        _sc_body,
        out_type=jax.ShapeDtypeStruct((LH, B // 4, 4, EMBED_DIM), jnp.float32),
        mesh=mesh,
        scratch_types=[
            pltpu.VMEM((2, CHUNK), jnp.int32),
            pltpu.VMEM((2, CHUNK, EMBED_DIM), jnp.float32),
            pltpu.SemaphoreType.DMA,
            pltpu.SemaphoreType.DMA,
        ],
        compiler_params=pltpu.CompilerParams(use_tc_tiling_on_sc=False),
    )(table, x_r)


WT = 16384
BRT = WT // 4
NWT = -(-NUM_TOKENS // WT)
PAD_TOKENS = NWT * WT


def _tc0_body(in_ref, out_ref, scratch):
    x = in_ref[...]
    for m in range(4):
        scratch[m * EMBED_DIM:(m + 1) * EMBED_DIM, :] = (
            x[:, m * BRT:(m + 1) * BRT])
    out_ref[...] = scratch[...].T


def _tc_relayout_table(table_t):
    return pl.pallas_call(
        _tc0_body,
        grid=(NWT,),
        in_specs=[pl.BlockSpec((EMBED_DIM, WT), lambda w: (0, w))],
        out_specs=pl.BlockSpec((BRT, 128), lambda w: (w, 0)),
        out_shape=jax.ShapeDtypeStruct((PAD_TOKENS // 4, 128), jnp.float32),
        scratch_shapes=[pltpu.VMEM((128, BRT), jnp.float32)],
    )(table_t)


def _tc_tanh_t(rows2, l_off, prev=None):
    def body(*refs):
        in_ref, out_ref = refs[0], refs[-1]
        y = jnp.tanh(in_ref[...]).T
        for m in range(4):
            out_ref[0, :, m * BBR:(m + 1) * BBR] = (
                y[m * EMBED_DIM:(m + 1) * EMBED_DIM, :])

    in_specs = [pl.BlockSpec((BBR, 128), lambda l, g: (l * NGB + g, 0))]
    args = [rows2]
    aliases = {}
    if prev is not None:
        in_specs.append(pl.BlockSpec(memory_space=pl.ANY))
        args.append(prev)
        aliases = {1: 0}
    return pl.pallas_call(
        body,
        grid=(LH, NGB),
        in_specs=in_specs,
        out_specs=pl.BlockSpec((1, EMBED_DIM, 4 * BBR),
                               lambda l, g: (l + l_off, 0, g)),
        out_shape=jax.ShapeDtypeStruct((L, EMBED_DIM, B), jnp.float32),
        input_output_aliases=aliases,
    )(*args)


@jax.jit
def kernel(x, table):
    xt = x.T
    rem = xt & (WT - 1)
    xt = (xt - rem) + ((rem & (BRT - 1)) << 2) + (rem >> (BRT.bit_length() - 1))
    table_rm = _tc_relayout_table(table.T).reshape(PAD_TOKENS, EMBED_DIM)
    t = None
    for h in range(2):
        x_h = xt[h * LH:(h + 1) * LH].reshape(NW, NCHUNK, CHUNK)
        rows = _sc_gather(table_rm, x_h)
        t = _tc_tanh_t(rows.reshape(TOTAL_H * EMBED_DIM // 128, 128),
                       h * LH, t)
    return t.transpose(2, 0, 1)


# speedup vs baseline: 7.4029x; 1.0046x over previous
import functools

import jax
import jax.numpy as jnp
from jax import lax
from jax.experimental import pallas as pl
from jax.experimental.pallas import tpu as pltpu
from jax.experimental.pallas import tpu_sc as plsc

NUM_TOKENS = 1000000
EMBED_DIM = 32
B = 16384
L = 50

NW = 32
CHUNK = 512
SUB = 128
NSUB = CHUNK // SUB
TOTAL = B * L
LH = 10
TOTAL_H = B * LH
PER_W = TOTAL_H // NW
NCHUNK = PER_W // CHUNK
NPAIR = NCHUNK // 2

BBR = 4096
NGB = B // (4 * BBR)


def _fire(table_hbm, idx_v, rows_v, buf, sem):
    for j in range(NSUB):
        pltpu.async_copy(
            table_hbm.at[idx_v.at[buf, pl.ds(j * SUB, SUB)]],
            rows_v.at[buf, pl.ds(j * SUB, SUB)],
            sem)


def _drain(table_hbm, idx_v, rows_v, buf, sem):
    for j in range(NSUB):
        pltpu.make_async_copy(
            table_hbm.at[idx_v.at[buf, pl.ds(j * SUB, SUB)]],
            rows_v.at[buf, pl.ds(j * SUB, SUB)],
            sem).wait()


def _sc_body(table_hbm, x_hbm, out_hbm, idx_v, rows_v, sem0, sem1):
    wid = lax.axis_index("s") * 2 + lax.axis_index("c")
    base = wid * PER_W

    def store(buf, c):
        p0 = base + c * CHUNK
        li = p0 >> 14
        r = p0 & (B - 1)
        m = r >> 12
        k0 = r & (B // 4 - 1)
        pltpu.sync_copy(rows_v.at[buf],
                        out_hbm.at[li, pl.ds(k0, CHUNK), m])

    pltpu.sync_copy(x_hbm.at[wid, 0], idx_v.at[0])
    _fire(table_hbm, idx_v, rows_v, 0, sem0)

    def pair(q, _):
        c1 = 2 * q + 1
        pltpu.sync_copy(x_hbm.at[wid, c1], idx_v.at[1])
        _fire(table_hbm, idx_v, rows_v, 1, sem1)
        _drain(table_hbm, idx_v, rows_v, 0, sem0)
        store(0, c1 - 1)

        if NCHUNK % 2 == 1:
            pltpu.sync_copy(x_hbm.at[wid, c1 + 1], idx_v.at[0])
            _fire(table_hbm, idx_v, rows_v, 0, sem0)
        else:
            @pl.when(q < NPAIR - 1)
            def _():
                pltpu.sync_copy(x_hbm.at[wid, c1 + 1], idx_v.at[0])
                _fire(table_hbm, idx_v, rows_v, 0, sem0)

        _drain(table_hbm, idx_v, rows_v, 1, sem1)
        store(1, c1)
        return 0

    lax.fori_loop(0, NPAIR, pair, 0)
    if NCHUNK % 2 == 1:
        _drain(table_hbm, idx_v, rows_v, 0, sem0)
        store(0, NCHUNK - 1)


def _sc_gather(table, x_r):
    mesh = plsc.VectorSubcoreMesh(core_axis_name="c", subcore_axis_name="s")
    return pl.kernel(
        _sc_body,
        out_type=jax.ShapeDtypeStruct((LH, B // 4, 4, EMBED_DIM), jnp.float32),
        mesh=mesh,
        scratch_types=[
            pltpu.VMEM((2, CHUNK), jnp.int32),
            pltpu.VMEM((2, CHUNK, EMBED_DIM), jnp.float32),
            pltpu.SemaphoreType.DMA,
            pltpu.SemaphoreType.DMA,
        ],
        compiler_params=pltpu.CompilerParams(use_tc_tiling_on_sc=False),
    )(table, x_r)


WT = 16384
BRT = WT // 4
NWT = -(-NUM_TOKENS // WT)
PAD_TOKENS = NWT * WT


def _tc0_body(in_ref, out_ref, scratch):
    x = in_ref[...]
    for m in range(4):
        scratch[m * EMBED_DIM:(m + 1) * EMBED_DIM, :] = (
            x[:, m * BRT:(m + 1) * BRT])
    out_ref[...] = scratch[...].T


def _tc_relayout_table(table_t):
    return pl.pallas_call(
        _tc0_body,
        grid=(NWT,),
        in_specs=[pl.BlockSpec((EMBED_DIM, WT), lambda w: (0, w))],
        out_specs=pl.BlockSpec((BRT, 128), lambda w: (w, 0)),
        out_shape=jax.ShapeDtypeStruct((PAD_TOKENS // 4, 128), jnp.float32),
        scratch_shapes=[pltpu.VMEM((128, BRT), jnp.float32)],
    )(table_t)


def _tc_tanh_t(rows2, l_off, prev=None):
    def body(*refs):
        in_ref, out_ref = refs[0], refs[-1]
        y = jnp.tanh(in_ref[...]).T
        for m in range(4):
            out_ref[0, :, m * BBR:(m + 1) * BBR] = (
                y[m * EMBED_DIM:(m + 1) * EMBED_DIM, :])

    in_specs = [pl.BlockSpec((BBR, 128), lambda l, g: (l * NGB + g, 0))]
    args = [rows2]
    aliases = {}
    if prev is not None:
        in_specs.append(pl.BlockSpec(memory_space=pl.ANY))
        args.append(prev)
        aliases = {1: 0}
    return pl.pallas_call(
        body,
        grid=(LH, NGB),
        in_specs=in_specs,
        out_specs=pl.BlockSpec((1, EMBED_DIM, 4 * BBR),
                               lambda l, g: (l + l_off, 0, g)),
        out_shape=jax.ShapeDtypeStruct((L, EMBED_DIM, B), jnp.float32),
        input_output_aliases=aliases,
    )(*args)


@jax.jit
def kernel(x, table):
    xt = x.T
    rem = xt & (WT - 1)
    xt = (xt - rem) + ((rem & (BRT - 1)) << 2) + (rem >> (BRT.bit_length() - 1))
    table_rm = _tc_relayout_table(table.T).reshape(PAD_TOKENS, EMBED_DIM)
    t = None
    for h in range(L // LH):
        x_h = xt[h * LH:(h + 1) * LH].reshape(NW, NCHUNK, CHUNK)
        rows = _sc_gather(table_rm, x_h)
        t = _tc_tanh_t(rows.reshape(TOTAL_H * EMBED_DIM // 128, 128),
                       h * LH, t)
    return t.transpose(2, 0, 1)
